# baseline (speedup 1.0000x reference)
import jax
import jax.numpy as jnp
from jax import lax
from jax.experimental import pallas as pl
from jax.experimental.pallas import tpu as pltpu
from jax.experimental.pallas import tpu_sc as plsc

N_NODES = 100000
IN_SIZE = 128
OUT_SIZE = 128
S = 10
B = 4096
M = B * (S + 1)

NW = 32
A_PW = B // NW
N_PW = (M - B) // NW
R_PW = A_PW + N_PW
CHUNK = 128
NCH = R_PW // CHUNK


def _sc_body(emb_hbm, nodes_hbm, neigh_hbm,
             self_s_hbm, agg_s_hbm, self_n_hbm, agg_n_hbm,
             idx_v, nidxraw, nidx_t0, nidx_t1, selfbuf, aggbuf,
             sem_g0, sem_g1, sem_w):
    wid = lax.axis_index("s") * 2 + lax.axis_index("c")
    sems_g = (sem_g0, sem_g1)
    nidx_t = (nidx_t0, nidx_t1)

    pltpu.sync_copy(nodes_hbm.at[pl.ds(wid * A_PW, A_PW)],
                    idx_v.at[pl.ds(0, A_PW)])
    pltpu.sync_copy(nodes_hbm.at[pl.ds(B + wid * N_PW, N_PW)],
                    idx_v.at[pl.ds(A_PW, N_PW)])
    pltpu.sync_copy(neigh_hbm.at[pl.ds(wid * A_PW * S, A_PW * S)],
                    nidxraw.at[pl.ds(0, A_PW * S)])
    pltpu.sync_copy(neigh_hbm.at[pl.ds((B + wid * N_PW) * S, N_PW * S)],
                    nidxraw.at[pl.ds(A_PW * S, N_PW * S)])

    zero16 = jnp.zeros((16,), jnp.float32)
    iota_s = lax.iota(jnp.int32, 16) * S

    def zero_agg(buf):
        def zrow(i, carry):
            for k in range(IN_SIZE // 16):
                aggbuf[buf, i, pl.ds(k * 16, 16)] = zero16
            return carry
        lax.fori_loop(0, CHUNK, zrow, 0)

    def transpose_idx(c, buf):
        dst = nidx_t[buf]

        def grp(g, carry):
            rowv = iota_s + ((c * CHUNK + g * 16) * S)
            for j in range(S):
                v = plsc.load_gather(nidxraw, [rowv + j])
                dst[pl.ds(j * CHUNK + g * 16, 16)] = v
            return carry
        lax.fori_loop(0, CHUNK // 16, grp, 0)

    def fire(c, buf):
        cps = [pltpu.async_copy(
            emb_hbm.at[idx_v.at[pl.ds(c * CHUNK, CHUNK)]],
            selfbuf.at[buf], sems_g[buf])]
        for j in range(S):
            cps.append(pltpu.async_copy(
                emb_hbm.at[nidx_t[buf].at[pl.ds(j * CHUNK, CHUNK)]],
                aggbuf.at[buf], sems_g[buf], add=True))
        return cps

    def out_refs(c):
        if c == 0:
            return self_s_hbm, agg_s_hbm, wid * A_PW
        return self_n_hbm, agg_n_hbm, wid * N_PW + (c - 1) * CHUNK

    zero_agg(0)
    transpose_idx(0, 0)
    gath = {0: fire(0, 0)}
    writes = {}
    for c in range(NCH):
        buf = c % 2
        nxt = 1 - buf
        if c >= 1:
            for cp in writes[c - 1]:
                cp.wait()
        if c + 1 < NCH:
            zero_agg(nxt)
            transpose_idx(c + 1, nxt)
            gath[c + 1] = fire(c + 1, nxt)
        for cp in gath[c]:
            cp.wait()
        o_self, o_agg, row = out_refs(c)
        writes[c] = [
            pltpu.async_copy(selfbuf.at[buf],
                             o_self.at[pl.ds(row, CHUNK), :], sem_w),
            pltpu.async_copy(aggbuf.at[buf],
                             o_agg.at[pl.ds(row, CHUNK), :], sem_w),
        ]
    for cp in writes[NCH - 1]:
        cp.wait()


def _sc_gather(emb_table, nodes1, neigh_flat):
    mesh = plsc.VectorSubcoreMesh(core_axis_name="c", subcore_axis_name="s")
    f = pl.kernel(
        _sc_body,
        compiler_params=pltpu.CompilerParams(needs_layout_passes=False),
        out_type=[
            jax.ShapeDtypeStruct((B, IN_SIZE), jnp.float32),
            jax.ShapeDtypeStruct((B, IN_SIZE), jnp.float32),
            jax.ShapeDtypeStruct((M - B, IN_SIZE), jnp.float32),
            jax.ShapeDtypeStruct((M - B, IN_SIZE), jnp.float32),
        ],
        mesh=mesh,
        scratch_types=[
            pltpu.VMEM((R_PW,), jnp.int32),
            pltpu.VMEM((R_PW * S,), jnp.int32),
            pltpu.VMEM((CHUNK * S,), jnp.int32),
            pltpu.VMEM((CHUNK * S,), jnp.int32),
            pltpu.VMEM((2, CHUNK, IN_SIZE), jnp.float32),
            pltpu.VMEM((2, CHUNK, IN_SIZE), jnp.float32),
            pltpu.SemaphoreType.DMA,
            pltpu.SemaphoreType.DMA,
            pltpu.SemaphoreType.DMA,
        ],
    )
    return f(emb_table, nodes1, neigh_flat)


def _tc_body(ss, sa, ns, na, w1a, w1b, w2a, w2b, o):
    f32 = jnp.float32
    h1s = jnp.maximum(
        jnp.dot(ss[:], w1a[:], preferred_element_type=f32)
        + jnp.dot(sa[:], w1b[:], preferred_element_type=f32), 0.0)
    h1n = jnp.maximum(
        jnp.dot(ns[:], w1a[:], preferred_element_type=f32)
        + jnp.dot(na[:], w1b[:], preferred_element_type=f32), 0.0)
    agg1 = jnp.sum(h1n.reshape(h1s.shape[0], S, OUT_SIZE), axis=1)
    o[:] = jnp.maximum(
        jnp.dot(h1s, w2a[:], preferred_element_type=f32)
        + jnp.dot(agg1, w2b[:], preferred_element_type=f32), 0.0)


_TB = 256
_TC_GRID = B // _TB


def _tc_call(self_s, agg_s, self_n, agg_n, w1a, w1b, w2a, w2b):
    wspec = pl.BlockSpec((IN_SIZE, OUT_SIZE), lambda p: (0, 0))
    return pl.pallas_call(
        _tc_body,
        grid=(_TC_GRID,),
        in_specs=[
            pl.BlockSpec((_TB, IN_SIZE), lambda p: (p, 0)),
            pl.BlockSpec((_TB, IN_SIZE), lambda p: (p, 0)),
            pl.BlockSpec((_TB * S, IN_SIZE), lambda p: (p, 0)),
            pl.BlockSpec((_TB * S, IN_SIZE), lambda p: (p, 0)),
            wspec, wspec, wspec, wspec,
        ],
        out_specs=pl.BlockSpec((_TB, OUT_SIZE), lambda p: (p, 0)),
        out_shape=jax.ShapeDtypeStruct((B, OUT_SIZE), jnp.float32),
    )(self_s, agg_s, self_n, agg_n, w1a, w1b, w2a, w2b)


def kernel(emb_table, W1, W2, node_batch, nodes1, neigh1, neigh2):
    neigh_flat = neigh1.reshape(-1)
    self_s, agg_s, self_n, agg_n = _sc_gather(emb_table, nodes1, neigh_flat)
    w1a = W1[:, :IN_SIZE].T
    w1b = W1[:, IN_SIZE:].T * (1.0 / S)
    w2a = W2[:, :OUT_SIZE].T
    w2b = W2[:, OUT_SIZE:].T * (1.0 / S)
    return _tc_call(self_s, agg_s, self_n, agg_n, w1a, w1b, w2a, w2b)


# speedup vs baseline: 1.1068x; 1.1068x over previous
import functools

import jax
import jax.numpy as jnp
from jax import lax
from jax.experimental import pallas as pl
from jax.experimental.pallas import tpu as pltpu
from jax.experimental.pallas import tpu_sc as plsc

N_NODES = 100000
IN_SIZE = 128
OUT_SIZE = 128
S = 10
B = 4096
M = B * (S + 1)

NW = 32
PHASES = 2
A_P = B // PHASES
N_P = A_P * S
A_PW = A_P // NW
N_PW = N_P // NW
R_PW = A_PW + N_PW
CHUNK = 128

_CHUNKS = [(0, A_PW, True, 0)]
_off, _row = A_PW, 0
while _row < N_PW:
    _CHUNKS.append((_off, CHUNK, False, _row))
    _off += CHUNK
    _row += CHUNK


def _sc_body(ph, emb_hbm, nodes_hbm, neight_hbm,
             self_s_hbm, agg_s_hbm, self_n_hbm, agg_n_hbm,
             idx_v, nidx_v, selfbuf, aggbuf, sem_g0, sem_g1, sem_w):
    wid = lax.axis_index("s") * 2 + lax.axis_index("c")
    sems_g = (sem_g0, sem_g1)

    a_col = ph * A_P + wid * A_PW
    n_col = B + ph * N_P + wid * N_PW
    pltpu.sync_copy(nodes_hbm.at[pl.ds(a_col, A_PW)],
                    idx_v.at[pl.ds(0, A_PW)])
    pltpu.sync_copy(nodes_hbm.at[pl.ds(n_col, N_PW)],
                    idx_v.at[pl.ds(A_PW, N_PW)])
    for j in range(S):
        pltpu.sync_copy(neight_hbm.at[pl.ds(j * M + a_col, A_PW)],
                        nidx_v.at[pl.ds(j * R_PW, A_PW)])
        pltpu.sync_copy(neight_hbm.at[pl.ds(j * M + n_col, N_PW)],
                        nidx_v.at[pl.ds(j * R_PW + A_PW, N_PW)])

    zero16 = jnp.zeros((16,), jnp.float32)

    def zero_agg(buf, size):
        def zrow(i, carry):
            for k in range(IN_SIZE // 16):
                aggbuf[buf, i, pl.ds(k * 16, 16)] = zero16
            return carry
        lax.fori_loop(0, size, zrow, 0)

    def fire(ch, buf):
        off, size, _, _ = ch
        cps = [pltpu.async_copy(
            emb_hbm.at[idx_v.at[pl.ds(off, size)]],
            selfbuf.at[buf, pl.ds(0, size), :], sems_g[buf])]
        for j in range(S):
            cps.append(pltpu.async_copy(
                emb_hbm.at[nidx_v.at[pl.ds(j * R_PW + off, size)]],
                aggbuf.at[buf, pl.ds(0, size), :], sems_g[buf], add=True))
        return cps

    nch = len(_CHUNKS)
    zero_agg(0, _CHUNKS[0][1])
    gath = {0: fire(_CHUNKS[0], 0)}
    writes = {}
    for c in range(nch):
        buf = c % 2
        nxt = 1 - buf
        if c >= 1:
            for cp in writes[c - 1]:
                cp.wait()
        if c + 1 < nch:
            zero_agg(nxt, _CHUNKS[c + 1][1])
            gath[c + 1] = fire(_CHUNKS[c + 1], nxt)
        for cp in gath[c]:
            cp.wait()
        _, size, is_a, lrow = _CHUNKS[c]
        if is_a:
            o_self, o_agg, row = self_s_hbm, agg_s_hbm, wid * A_PW + lrow
        else:
            o_self, o_agg, row = self_n_hbm, agg_n_hbm, wid * N_PW + lrow
        writes[c] = [
            pltpu.async_copy(selfbuf.at[buf, pl.ds(0, size), :],
                             o_self.at[pl.ds(row, size), :], sem_w),
            pltpu.async_copy(aggbuf.at[buf, pl.ds(0, size), :],
                             o_agg.at[pl.ds(row, size), :], sem_w),
        ]
    for cp in writes[nch - 1]:
        cp.wait()


def _make_sc(ph):
    mesh = plsc.VectorSubcoreMesh(core_axis_name="c", subcore_axis_name="s")
    return pl.kernel(
        functools.partial(_sc_body, ph),
        out_type=[
            jax.ShapeDtypeStruct((A_P, IN_SIZE), jnp.float32),
            jax.ShapeDtypeStruct((A_P, IN_SIZE), jnp.float32),
            jax.ShapeDtypeStruct((N_P, IN_SIZE), jnp.float32),
            jax.ShapeDtypeStruct((N_P, IN_SIZE), jnp.float32),
        ],
        mesh=mesh,
        scratch_types=[
            pltpu.VMEM((R_PW,), jnp.int32),
            pltpu.VMEM((S * R_PW,), jnp.int32),
            pltpu.VMEM((2, CHUNK, IN_SIZE), jnp.float32),
            pltpu.VMEM((2, CHUNK, IN_SIZE), jnp.float32),
            pltpu.SemaphoreType.DMA,
            pltpu.SemaphoreType.DMA,
            pltpu.SemaphoreType.DMA,
        ],
    )


def _tc_body(ss, sa, ns, na, w1a, w1b, w2a, w2b, o):
    f32 = jnp.float32
    h1s = jnp.maximum(
        jnp.dot(ss[:], w1a[:], preferred_element_type=f32)
        + jnp.dot(sa[:], w1b[:], preferred_element_type=f32), 0.0)
    h1n = jnp.maximum(
        jnp.dot(ns[:], w1a[:], preferred_element_type=f32)
        + jnp.dot(na[:], w1b[:], preferred_element_type=f32), 0.0)
    agg1 = jnp.sum(h1n.reshape(h1s.shape[0], S, OUT_SIZE), axis=1)
    o[:] = jnp.maximum(
        jnp.dot(h1s, w2a[:], preferred_element_type=f32)
        + jnp.dot(agg1, w2b[:], preferred_element_type=f32), 0.0)


_TB = 256
_TC_GRID = A_P // _TB


def _tc_call(self_s, agg_s, self_n, agg_n, w1a, w1b, w2a, w2b):
    wspec = pl.BlockSpec((IN_SIZE, OUT_SIZE), lambda p: (0, 0))
    return pl.pallas_call(
        _tc_body,
        grid=(_TC_GRID,),
        in_specs=[
            pl.BlockSpec((_TB, IN_SIZE), lambda p: (p, 0)),
            pl.BlockSpec((_TB, IN_SIZE), lambda p: (p, 0)),
            pl.BlockSpec((_TB * S, IN_SIZE), lambda p: (p, 0)),
            pl.BlockSpec((_TB * S, IN_SIZE), lambda p: (p, 0)),
            wspec, wspec, wspec, wspec,
        ],
        out_specs=pl.BlockSpec((_TB, OUT_SIZE), lambda p: (p, 0)),
        out_shape=jax.ShapeDtypeStruct((A_P, OUT_SIZE), jnp.float32),
    )(self_s, agg_s, self_n, agg_n, w1a, w1b, w2a, w2b)


def kernel(emb_table, W1, W2, node_batch, nodes1, neigh1, neigh2):
    neigh_t = neigh1.T.reshape(-1)
    w1a = W1[:, :IN_SIZE].T
    w1b = W1[:, IN_SIZE:].T * (1.0 / S)
    w2a = W2[:, :OUT_SIZE].T
    w2b = W2[:, OUT_SIZE:].T * (1.0 / S)
    outs = []
    for ph in range(PHASES):
        self_s, agg_s, self_n, agg_n = _make_sc(ph)(
            emb_table, nodes1, neigh_t)
        outs.append(_tc_call(self_s, agg_s, self_n, agg_n,
                             w1a, w1b, w2a, w2b))
    return jnp.concatenate(outs, axis=0)


# speedup vs baseline: 1.2240x; 1.1059x over previous
import functools

import jax
import jax.numpy as jnp
from jax import lax
from jax.experimental import pallas as pl
from jax.experimental.pallas import tpu as pltpu
from jax.experimental.pallas import tpu_sc as plsc

N_NODES = 100000
IN_SIZE = 128
OUT_SIZE = 128
S = 10
B = 4096
M = B * (S + 1)

NW = 32
PHASES = 2
A_P = B // PHASES
N_P = A_P * S
A_PW = A_P // NW
N_PW = N_P // NW
R_PW = A_PW + N_PW
CHUNK = 128

_CHUNKS = [(0, A_PW, True, 0)]
_off, _row = A_PW, 0
while _row < N_PW:
    _CHUNKS.append((_off, CHUNK, False, _row))
    _off += CHUNK
    _row += CHUNK


def _sc_body(ph, emb_hbm, nodes_hbm, neight_hbm,
             self_s_hbm, agg_s_hbm, self_n_hbm, agg_n_hbm,
             idx_v, nidx_v, selfbuf, aggbuf, sem_g0, sem_g1, sem_w):
    wid = lax.axis_index("s") * 2 + lax.axis_index("c")
    sems_g = (sem_g0, sem_g1)

    a_col = ph * A_P + wid * A_PW
    n_col = B + ph * N_P + wid * N_PW
    stg = [
        pltpu.async_copy(nodes_hbm.at[pl.ds(a_col, A_PW)],
                         idx_v.at[pl.ds(0, A_PW)], sem_w),
        pltpu.async_copy(nodes_hbm.at[pl.ds(n_col, N_PW)],
                         idx_v.at[pl.ds(A_PW, N_PW)], sem_w),
    ]
    for j in range(S):
        stg.append(pltpu.async_copy(neight_hbm.at[pl.ds(j * M + a_col, A_PW)],
                                    nidx_v.at[pl.ds(j * R_PW, A_PW)], sem_w))
        stg.append(pltpu.async_copy(neight_hbm.at[pl.ds(j * M + n_col, N_PW)],
                                    nidx_v.at[pl.ds(j * R_PW + A_PW, N_PW)],
                                    sem_w))
    for cp in stg:
        cp.wait()

    zero16 = jnp.zeros((16,), jnp.float32)

    def zero_agg(buf, size):
        def zrow(i, carry):
            for k in range(IN_SIZE // 16):
                aggbuf[buf, i, pl.ds(k * 16, 16)] = zero16
            return carry
        lax.fori_loop(0, size, zrow, 0)

    def fire(ch, buf):
        off, size, _, _ = ch
        cps = [pltpu.async_copy(
            emb_hbm.at[idx_v.at[pl.ds(off, size)]],
            selfbuf.at[buf, pl.ds(0, size), :], sems_g[buf])]
        for j in range(S):
            cps.append(pltpu.async_copy(
                emb_hbm.at[nidx_v.at[pl.ds(j * R_PW + off, size)]],
                aggbuf.at[buf, pl.ds(0, size), :], sems_g[buf], add=True))
        return cps

    nch = len(_CHUNKS)
    zero_agg(0, _CHUNKS[0][1])
    gath = {0: fire(_CHUNKS[0], 0)}
    writes = {}
    for c in range(nch):
        buf = c % 2
        nxt = 1 - buf
        if c >= 1:
            for cp in writes[c - 1]:
                cp.wait()
        if c + 1 < nch:
            zero_agg(nxt, _CHUNKS[c + 1][1])
            gath[c + 1] = fire(_CHUNKS[c + 1], nxt)
        for cp in gath[c]:
            cp.wait()
        _, size, is_a, lrow = _CHUNKS[c]
        if is_a:
            o_self, o_agg, row = self_s_hbm, agg_s_hbm, wid * A_PW + lrow
        else:
            o_self, o_agg, row = self_n_hbm, agg_n_hbm, wid * N_PW + lrow
        writes[c] = [
            pltpu.async_copy(selfbuf.at[buf, pl.ds(0, size), :],
                             o_self.at[pl.ds(row, size), :], sem_w),
            pltpu.async_copy(aggbuf.at[buf, pl.ds(0, size), :],
                             o_agg.at[pl.ds(row, size), :], sem_w),
        ]
    for cp in writes[nch - 1]:
        cp.wait()


def _make_sc(ph):
    mesh = plsc.VectorSubcoreMesh(core_axis_name="c", subcore_axis_name="s")
    return pl.kernel(
        functools.partial(_sc_body, ph),
        out_type=[
            jax.ShapeDtypeStruct((A_P, IN_SIZE), jnp.float32),
            jax.ShapeDtypeStruct((A_P, IN_SIZE), jnp.float32),
            jax.ShapeDtypeStruct((N_P, IN_SIZE), jnp.float32),
            jax.ShapeDtypeStruct((N_P, IN_SIZE), jnp.float32),
        ],
        mesh=mesh,
        scratch_types=[
            pltpu.VMEM((R_PW,), jnp.int32),
            pltpu.VMEM((S * R_PW,), jnp.int32),
            pltpu.VMEM((2, CHUNK, IN_SIZE), jnp.float32),
            pltpu.VMEM((2, CHUNK, IN_SIZE), jnp.float32),
            pltpu.SemaphoreType.DMA,
            pltpu.SemaphoreType.DMA,
            pltpu.SemaphoreType.DMA,
        ],
    )


def _tc_body(ss, sa, ns, na, w1a, w1b, w2a, w2b, o):
    f32 = jnp.float32
    h1s = jnp.maximum(
        jnp.dot(ss[:], w1a[:], preferred_element_type=f32)
        + jnp.dot(sa[:], w1b[:], preferred_element_type=f32), 0.0)
    h1n = jnp.maximum(
        jnp.dot(ns[:], w1a[:], preferred_element_type=f32)
        + jnp.dot(na[:], w1b[:], preferred_element_type=f32), 0.0)
    agg1 = jnp.sum(h1n.reshape(h1s.shape[0], S, OUT_SIZE), axis=1)
    o[:] = jnp.maximum(
        jnp.dot(h1s, w2a[:], preferred_element_type=f32)
        + jnp.dot(agg1, w2b[:], preferred_element_type=f32), 0.0)


_TB = 256
_TC_GRID = A_P // _TB


def _tc_call(self_s, agg_s, self_n, agg_n, w1a, w1b, w2a, w2b):
    wspec = pl.BlockSpec((IN_SIZE, OUT_SIZE), lambda p: (0, 0))
    return pl.pallas_call(
        _tc_body,
        grid=(_TC_GRID,),
        in_specs=[
            pl.BlockSpec((_TB, IN_SIZE), lambda p: (p, 0)),
            pl.BlockSpec((_TB, IN_SIZE), lambda p: (p, 0)),
            pl.BlockSpec((_TB * S, IN_SIZE), lambda p: (p, 0)),
            pl.BlockSpec((_TB * S, IN_SIZE), lambda p: (p, 0)),
            wspec, wspec, wspec, wspec,
        ],
        out_specs=pl.BlockSpec((_TB, OUT_SIZE), lambda p: (p, 0)),
        out_shape=jax.ShapeDtypeStruct((A_P, OUT_SIZE), jnp.float32),
    )(self_s, agg_s, self_n, agg_n, w1a, w1b, w2a, w2b)


def kernel(emb_table, W1, W2, node_batch, nodes1, neigh1, neigh2):
    neigh_t = neigh1.T.reshape(-1)
    w1a = W1[:, :IN_SIZE].T
    w1b = W1[:, IN_SIZE:].T * (1.0 / S)
    w2a = W2[:, :OUT_SIZE].T
    w2b = W2[:, OUT_SIZE:].T * (1.0 / S)
    outs = []
    for ph in range(PHASES):
        self_s, agg_s, self_n, agg_n = _make_sc(ph)(
            emb_table, nodes1, neigh_t)
        outs.append(_tc_call(self_s, agg_s, self_n, agg_n,
                             w1a, w1b, w2a, w2b))
    return jnp.concatenate(outs, axis=0)
